# baseline (speedup 1.0000x reference)
import jax
import jax.numpy as jnp
from jax import lax
from jax.experimental import pallas as pl
from jax.experimental.pallas import tpu as pltpu
from jax.experimental.pallas import tpu_sc as plsc

VOCAB = 1000000
D = 64
B = 16384
CTX = 4
NEG = 5

NC = 2
NS = 16
NW = NC * NS
B_PER_W = B // NW
CHUNK = 64
NCHUNK = B_PER_W // CHUNK
GROUPS = CHUNK // 16

NCI = CHUNK * CTX
NNI = CHUNK * NEG

VB = VOCAB // 128
VHALF = VOCAB // 2


def _tr_block(tin, tout, lanes, width):
  fifteen = jnp.int32(15)
  one = jnp.int32(1)

  def s_body(s, _):
    w = jnp.bitwise_and(s + lanes, fifteen)

    def t_body(t, _):
      v = t * 16 + w
      rowv = lax.shift_right_logical(v, one)
      parc = lax.shift_left(jnp.bitwise_and(v, one), jnp.int32(6))
      for m in range(4):
        dvec = m * 16 + lanes
        x = plsc.load_gather(tin, [dvec, v])
        plsc.store_scatter(tout, [rowv, parc + dvec], x)
      return 0

    lax.fori_loop(0, width // 16, t_body, 0)
    return 0

  lax.fori_loop(0, 16, s_body, 0)


def _t_body(ctx_t_hbm, cen_t_hbm, ctx_tail_hbm, cen_tail_hbm,
            ctx2_hbm, cen2_hbm, tin, tout, sem):
  wid = lax.axis_index("s") * NC + lax.axis_index("c")
  lanes = lax.iota(jnp.int32, 16)

  nfull = VB // NW
  extra = VB - nfull * NW

  for src, tail, dst in ((ctx_t_hbm, ctx_tail_hbm, ctx2_hbm),
                         (cen_t_hbm, cen_tail_hbm, cen2_hbm)):
    def blk(j):
      pltpu.make_async_copy(src.at[:, pl.ds(j * 128, 128)], tin, sem).start()
      pltpu.make_async_copy(src.at[:, pl.ds(j * 128, 128)], tin, sem).wait()
      _tr_block(tin, tout, lanes, 128)
      pltpu.make_async_copy(tout, dst.at[pl.ds(j * 64, 64), :], sem).start()
      pltpu.make_async_copy(tout, dst.at[pl.ds(j * 64, 64), :], sem).wait()

    def loop_body(i, _):
      blk(wid + i * NW)
      return 0
    lax.fori_loop(0, nfull, loop_body, 0)

    @pl.when(wid < extra)
    def _():
      blk(nfull * NW + wid)

    @pl.when(wid == extra)
    def _():
      pltpu.make_async_copy(tail, tout.at[pl.ds(0, 32), :], sem).start()
      pltpu.make_async_copy(tail, tout.at[pl.ds(0, 32), :], sem).wait()
      pltpu.make_async_copy(
          tout.at[pl.ds(0, 32), :], dst.at[pl.ds(VB * 64, 32), :], sem).start()
      pltpu.make_async_copy(
          tout.at[pl.ds(0, 32), :], dst.at[pl.ds(VB * 64, 32), :], sem).wait()


def _body(ctx_idx_hbm, cen_idx_hbm, neg_idx_hbm, ctx_emb_hbm, cen_emb_hbm,
          pos_hbm, neg_hbm,
          idx_ctx, idx_cen, idx_neg, pr_ctx, pr_cen, pr_neg,
          rows_ctx, rows_cen, rows_neg, pos_v, neg_v, sem):
  wid = lax.axis_index("s") * NC + lax.axis_index("c")
  base = wid * B_PER_W

  lanes = lax.iota(jnp.int32, 16)
  one = jnp.int32(1)

  for c in range(NCHUNK):
    b0 = base + c * CHUNK
    pltpu.sync_copy(ctx_idx_hbm.at[pl.ds(b0 * CTX, NCI)], idx_ctx)
    pltpu.sync_copy(cen_idx_hbm.at[pl.ds(b0, CHUNK)], idx_cen)
    pltpu.sync_copy(neg_idx_hbm.at[pl.ds(b0 * NEG, NNI)], idx_neg)

    def shift_into(dst, src, n):
      def sbody(k, _):
        dst[pl.ds(k * 16, 16)] = lax.shift_right_logical(
            src[pl.ds(k * 16, 16)], one)
        return 0
      lax.fori_loop(0, n // 16, sbody, 0)
    shift_into(pr_ctx, idx_ctx, NCI)
    shift_into(pr_cen, idx_cen, CHUNK)
    shift_into(pr_neg, idx_neg, NNI)

    cps = []
    for k in range(NCI // 128):
      cps.append(pltpu.make_async_copy(
          ctx_emb_hbm.at[pr_ctx.at[pl.ds(k * 128, 128)]],
          rows_ctx.at[pl.ds(k * 128, 128)], sem))
    cps.append(pltpu.make_async_copy(
        cen_emb_hbm.at[pr_cen], rows_cen, sem))
    for k in range(NNI // 64):
      cps.append(pltpu.make_async_copy(
          cen_emb_hbm.at[pr_neg.at[pl.ds(k * 64, 64)]],
          rows_neg.at[pl.ds(k * 64, 64)], sem))
    for cp in cps:
      cp.start()
    for cp in cps:
      cp.wait()

    def group_body(g, _):
      bl = g * 16 + lanes
      ctx_rows = bl * CTX
      neg_rows = bl * NEG

      def half(iref, pos_vec):
        v = plsc.load_gather(iref, [pos_vec])
        return lax.shift_left(jnp.bitwise_and(v, one), jnp.int32(6))

      cb_c0 = half(idx_ctx, ctx_rows)
      cb_c1 = half(idx_ctx, ctx_rows + 1)
      cb_c2 = half(idx_ctx, ctx_rows + 2)
      cb_c3 = half(idx_ctx, ctx_rows + 3)
      cb_u = half(idx_cen, bl)
      cb_n0 = half(idx_neg, neg_rows)
      cb_n1 = half(idx_neg, neg_rows + 1)
      cb_n2 = half(idx_neg, neg_rows + 2)
      cb_n3 = half(idx_neg, neg_rows + 3)
      cb_n4 = half(idx_neg, neg_rows + 4)

      def d_body(d, acc):
        pos_a, n0, n1, n2, n3, n4 = acc
        v = plsc.load_gather(rows_ctx, [ctx_rows, cb_c0 + d])
        v = v + plsc.load_gather(rows_ctx, [ctx_rows + 1, cb_c1 + d])
        v = v + plsc.load_gather(rows_ctx, [ctx_rows + 2, cb_c2 + d])
        v = v + plsc.load_gather(rows_ctx, [ctx_rows + 3, cb_c3 + d])
        u = plsc.load_gather(rows_cen, [bl, cb_u + d])
        pos_a = pos_a + v * u
        n0 = n0 + v * plsc.load_gather(rows_neg, [neg_rows, cb_n0 + d])
        n1 = n1 + v * plsc.load_gather(rows_neg, [neg_rows + 1, cb_n1 + d])
        n2 = n2 + v * plsc.load_gather(rows_neg, [neg_rows + 2, cb_n2 + d])
        n3 = n3 + v * plsc.load_gather(rows_neg, [neg_rows + 3, cb_n3 + d])
        n4 = n4 + v * plsc.load_gather(rows_neg, [neg_rows + 4, cb_n4 + d])
        return pos_a, n0, n1, n2, n3, n4

      z = jnp.zeros((16,), jnp.float32)
      pos_a, n0, n1, n2, n3, n4 = lax.fori_loop(
          0, D, d_body, (z, z, z, z, z, z))

      quarter = jnp.float32(0.25)
      pos_v[pl.ds(g * 16, 16)] = pos_a * quarter
      plsc.store_scatter(neg_v, [neg_rows], n0 * quarter)
      plsc.store_scatter(neg_v, [neg_rows + 1], n1 * quarter)
      plsc.store_scatter(neg_v, [neg_rows + 2], n2 * quarter)
      plsc.store_scatter(neg_v, [neg_rows + 3], n3 * quarter)
      plsc.store_scatter(neg_v, [neg_rows + 4], n4 * quarter)
      return 0

    lax.fori_loop(0, GROUPS, group_body, 0)

    pltpu.sync_copy(pos_v, pos_hbm.at[pl.ds(b0, CHUNK)])
    pltpu.sync_copy(neg_v, neg_hbm.at[pl.ds(b0 * NEG, NNI)])


@jax.jit
def _cbow_sc(ctx_idx, cen_idx, neg_idx, ctx_t, cen_t):
  mesh = plsc.VectorSubcoreMesh(core_axis_name="c", subcore_axis_name="s")
  tr = pl.kernel(
      _t_body,
      out_type=(
          jax.ShapeDtypeStruct((VHALF, 128), jnp.float32),
          jax.ShapeDtypeStruct((VHALF, 128), jnp.float32),
      ),
      mesh=mesh,
      compiler_params=pltpu.CompilerParams(
          needs_layout_passes=False, use_tc_tiling_on_sc=True),
      scratch_types=[
          pltpu.VMEM((D, 128), jnp.float32),
          pltpu.VMEM((D, 128), jnp.float32),
          pltpu.SemaphoreType.DMA,
      ],
  )
  ctx_tail = ctx_t[:, VB * 128:].T.reshape(32, 128)
  cen_tail = cen_t[:, VB * 128:].T.reshape(32, 128)
  ctx_emb2, cen_emb2 = tr(ctx_t, cen_t, ctx_tail, cen_tail)

  kfn = pl.kernel(
      _body,
      out_type=(
          jax.ShapeDtypeStruct((B,), jnp.float32),
          jax.ShapeDtypeStruct((B * NEG,), jnp.float32),
      ),
      mesh=mesh,
      compiler_params=pltpu.CompilerParams(
          needs_layout_passes=False, use_tc_tiling_on_sc=True),
      scratch_types=[
          pltpu.VMEM((NCI,), jnp.int32),
          pltpu.VMEM((CHUNK,), jnp.int32),
          pltpu.VMEM((NNI,), jnp.int32),
          pltpu.VMEM((NCI,), jnp.int32),
          pltpu.VMEM((CHUNK,), jnp.int32),
          pltpu.VMEM((NNI,), jnp.int32),
          pltpu.VMEM((NCI, 128), jnp.float32),
          pltpu.VMEM((CHUNK, 128), jnp.float32),
          pltpu.VMEM((NNI, 128), jnp.float32),
          pltpu.VMEM((CHUNK,), jnp.float32),
          pltpu.VMEM((NNI,), jnp.float32),
          pltpu.SemaphoreType.DMA,
      ],
  )
  return kfn(ctx_idx, cen_idx, neg_idx, ctx_emb2, cen_emb2)


def kernel(context_words, center_words, negative_samples, context_emb,
           center_emb):
  ctx_idx = context_words.reshape(-1).astype(jnp.int32)
  cen_idx = center_words.astype(jnp.int32)
  neg_idx = negative_samples.reshape(-1).astype(jnp.int32)
  pos, neg = _cbow_sc(ctx_idx, cen_idx, neg_idx, context_emb.T, center_emb.T)
  return pos, neg.reshape(B, NEG)


# speedup vs baseline: 1.5754x; 1.5754x over previous
import jax
import jax.numpy as jnp
from jax import lax
from jax.experimental import pallas as pl
from jax.experimental.pallas import tpu as pltpu
from jax.experimental.pallas import tpu_sc as plsc

VOCAB = 1000000
D = 64
B = 16384
CTX = 4
NEG = 5

NC = 2
NS = 16
NW = NC * NS
B_PER_W = B // NW
CHUNK = 64
NCHUNK = B_PER_W // CHUNK
GROUPS = CHUNK // 16

NCI = CHUNK * CTX
NNI = CHUNK * NEG

VB = VOCAB // 128
VHALF = VOCAB // 2


def _tr_block(tin, tout, lanes):
  fifteen = jnp.int32(15)
  one = jnp.int32(1)
  six = jnp.int32(6)
  dvecs = [m * 16 + lanes for m in range(4)]

  def s_body(s, _):
    w = jnp.bitwise_and(s + lanes, fifteen)
    for t in range(8):
      v = t * 16 + w
      rowv = lax.shift_right_logical(v, one)
      parc = lax.shift_left(jnp.bitwise_and(v, one), six)
      for m in range(4):
        x = plsc.load_gather(tin, [dvecs[m], v])
        plsc.store_scatter(tout, [rowv, parc + dvecs[m]], x)
    return 0

  lax.fori_loop(0, 16, s_body, 0)


def _t_body(ctx_t_hbm, cen_t_hbm, ctx_tail_hbm, cen_tail_hbm,
            ctx2_hbm, cen2_hbm,
            tin0, tin1, tout0, tout1, si0, si1, so0, so1, sem):
  wid = lax.axis_index("s") * NC + lax.axis_index("c")
  lanes = lax.iota(jnp.int32, 16)

  nfull = VB // NW
  npairs = nfull // 2
  extra = VB - nfull * NW

  for src, tail, dst in ((ctx_t_hbm, ctx_tail_hbm, ctx2_hbm),
                         (cen_t_hbm, cen_tail_hbm, cen2_hbm)):
    def start_in(j, buf, s):
      pltpu.make_async_copy(src.at[:, pl.ds(j * 128, 128)], buf, s).start()

    def wait_in(buf, s):
      pltpu.make_async_copy(src.at[:, pl.ds(0, 128)], buf, s).wait()

    def start_out(j, buf, s):
      pltpu.make_async_copy(buf, dst.at[pl.ds(j * 64, 64), :], s).start()

    def wait_out(buf, s):
      pltpu.make_async_copy(buf, dst.at[pl.ds(0, 64), :], s).wait()

    start_in(wid, tin0, si0)

    def pair_body(i2, _):
      jA = wid + (2 * i2) * NW
      jB = jA + NW
      wait_in(tin0, si0)
      start_in(jB, tin1, si1)

      @pl.when(i2 > 0)
      def _():
        wait_out(tout0, so0)
      _tr_block(tin0, tout0, lanes)
      start_out(jA, tout0, so0)

      wait_in(tin1, si1)

      @pl.when(i2 + 1 < npairs)
      def _():
        start_in(jA + 2 * NW, tin0, si0)

      @pl.when(i2 > 0)
      def _():
        wait_out(tout1, so1)
      _tr_block(tin1, tout1, lanes)
      start_out(jB, tout1, so1)
      return 0

    lax.fori_loop(0, npairs, pair_body, 0)
    wait_out(tout0, so0)
    wait_out(tout1, so1)

    @pl.when(wid < extra)
    def _():
      j = nfull * NW + wid
      pltpu.make_async_copy(src.at[:, pl.ds(j * 128, 128)], tin0, sem).start()
      pltpu.make_async_copy(src.at[:, pl.ds(j * 128, 128)], tin0, sem).wait()
      _tr_block(tin0, tout0, lanes)
      pltpu.make_async_copy(tout0, dst.at[pl.ds(j * 64, 64), :], sem).start()
      pltpu.make_async_copy(tout0, dst.at[pl.ds(j * 64, 64), :], sem).wait()

    @pl.when(wid == extra)
    def _():
      pltpu.make_async_copy(tail, tout1.at[pl.ds(0, 32), :], sem).start()
      pltpu.make_async_copy(tail, tout1.at[pl.ds(0, 32), :], sem).wait()
      pltpu.make_async_copy(
          tout1.at[pl.ds(0, 32), :], dst.at[pl.ds(VB * 64, 32), :],
          sem).start()
      pltpu.make_async_copy(
          tout1.at[pl.ds(0, 32), :], dst.at[pl.ds(VB * 64, 32), :],
          sem).wait()


def _body(ctx_idx_hbm, cen_idx_hbm, neg_idx_hbm, ctx_emb_hbm, cen_emb_hbm,
          pos_hbm, neg_hbm,
          idx_ctx, idx_cen, idx_neg, pr_ctx, pr_cen, pr_neg,
          rows_ctx, rows_cen, rows_neg, pos_v, neg_v, sem):
  wid = lax.axis_index("s") * NC + lax.axis_index("c")
  base = wid * B_PER_W

  lanes = lax.iota(jnp.int32, 16)
  one = jnp.int32(1)

  for c in range(NCHUNK):
    b0 = base + c * CHUNK
    pltpu.sync_copy(ctx_idx_hbm.at[pl.ds(b0 * CTX, NCI)], idx_ctx)
    pltpu.sync_copy(cen_idx_hbm.at[pl.ds(b0, CHUNK)], idx_cen)
    pltpu.sync_copy(neg_idx_hbm.at[pl.ds(b0 * NEG, NNI)], idx_neg)

    def shift_into(dst, src, n):
      def sbody(k, _):
        dst[pl.ds(k * 16, 16)] = lax.shift_right_logical(
            src[pl.ds(k * 16, 16)], one)
        return 0
      lax.fori_loop(0, n // 16, sbody, 0)
    shift_into(pr_ctx, idx_ctx, NCI)
    shift_into(pr_cen, idx_cen, CHUNK)
    shift_into(pr_neg, idx_neg, NNI)

    cps = []
    for k in range(NCI // 128):
      cps.append(pltpu.make_async_copy(
          ctx_emb_hbm.at[pr_ctx.at[pl.ds(k * 128, 128)]],
          rows_ctx.at[pl.ds(k * 128, 128)], sem))
    cps.append(pltpu.make_async_copy(
        cen_emb_hbm.at[pr_cen], rows_cen, sem))
    for k in range(NNI // 64):
      cps.append(pltpu.make_async_copy(
          cen_emb_hbm.at[pr_neg.at[pl.ds(k * 64, 64)]],
          rows_neg.at[pl.ds(k * 64, 64)], sem))
    for cp in cps:
      cp.start()
    for cp in cps:
      cp.wait()

    def group_body(g, _):
      bl = g * 16 + lanes
      ctx_rows = bl * CTX
      neg_rows = bl * NEG

      def half(iref, pos_vec):
        v = plsc.load_gather(iref, [pos_vec])
        return lax.shift_left(jnp.bitwise_and(v, one), jnp.int32(6))

      cb_c0 = half(idx_ctx, ctx_rows)
      cb_c1 = half(idx_ctx, ctx_rows + 1)
      cb_c2 = half(idx_ctx, ctx_rows + 2)
      cb_c3 = half(idx_ctx, ctx_rows + 3)
      cb_u = half(idx_cen, bl)
      cb_n0 = half(idx_neg, neg_rows)
      cb_n1 = half(idx_neg, neg_rows + 1)
      cb_n2 = half(idx_neg, neg_rows + 2)
      cb_n3 = half(idx_neg, neg_rows + 3)
      cb_n4 = half(idx_neg, neg_rows + 4)

      def d_body(d, acc):
        pos_a, n0, n1, n2, n3, n4 = acc
        v = plsc.load_gather(rows_ctx, [ctx_rows, cb_c0 + d])
        v = v + plsc.load_gather(rows_ctx, [ctx_rows + 1, cb_c1 + d])
        v = v + plsc.load_gather(rows_ctx, [ctx_rows + 2, cb_c2 + d])
        v = v + plsc.load_gather(rows_ctx, [ctx_rows + 3, cb_c3 + d])
        u = plsc.load_gather(rows_cen, [bl, cb_u + d])
        pos_a = pos_a + v * u
        n0 = n0 + v * plsc.load_gather(rows_neg, [neg_rows, cb_n0 + d])
        n1 = n1 + v * plsc.load_gather(rows_neg, [neg_rows + 1, cb_n1 + d])
        n2 = n2 + v * plsc.load_gather(rows_neg, [neg_rows + 2, cb_n2 + d])
        n3 = n3 + v * plsc.load_gather(rows_neg, [neg_rows + 3, cb_n3 + d])
        n4 = n4 + v * plsc.load_gather(rows_neg, [neg_rows + 4, cb_n4 + d])
        return pos_a, n0, n1, n2, n3, n4

      z = jnp.zeros((16,), jnp.float32)
      pos_a, n0, n1, n2, n3, n4 = lax.fori_loop(
          0, D, d_body, (z, z, z, z, z, z))

      quarter = jnp.float32(0.25)
      pos_v[pl.ds(g * 16, 16)] = pos_a * quarter
      plsc.store_scatter(neg_v, [neg_rows], n0 * quarter)
      plsc.store_scatter(neg_v, [neg_rows + 1], n1 * quarter)
      plsc.store_scatter(neg_v, [neg_rows + 2], n2 * quarter)
      plsc.store_scatter(neg_v, [neg_rows + 3], n3 * quarter)
      plsc.store_scatter(neg_v, [neg_rows + 4], n4 * quarter)
      return 0

    lax.fori_loop(0, GROUPS, group_body, 0)

    pltpu.sync_copy(pos_v, pos_hbm.at[pl.ds(b0, CHUNK)])
    pltpu.sync_copy(neg_v, neg_hbm.at[pl.ds(b0 * NEG, NNI)])


@jax.jit
def _cbow_sc(ctx_idx, cen_idx, neg_idx, ctx_t, cen_t):
  mesh = plsc.VectorSubcoreMesh(core_axis_name="c", subcore_axis_name="s")
  tr = pl.kernel(
      _t_body,
      out_type=(
          jax.ShapeDtypeStruct((VHALF, 128), jnp.float32),
          jax.ShapeDtypeStruct((VHALF, 128), jnp.float32),
      ),
      mesh=mesh,
      compiler_params=pltpu.CompilerParams(
          needs_layout_passes=False, use_tc_tiling_on_sc=True),
      scratch_types=[
          pltpu.VMEM((D, 128), jnp.float32),
          pltpu.VMEM((D, 128), jnp.float32),
          pltpu.VMEM((D, 128), jnp.float32),
          pltpu.VMEM((D, 128), jnp.float32),
          pltpu.SemaphoreType.DMA,
          pltpu.SemaphoreType.DMA,
          pltpu.SemaphoreType.DMA,
          pltpu.SemaphoreType.DMA,
          pltpu.SemaphoreType.DMA,
      ],
  )
  ctx_tail = ctx_t[:, VB * 128:].T.reshape(32, 128)
  cen_tail = cen_t[:, VB * 128:].T.reshape(32, 128)
  ctx_emb2, cen_emb2 = tr(ctx_t, cen_t, ctx_tail, cen_tail)

  kfn = pl.kernel(
      _body,
      out_type=(
          jax.ShapeDtypeStruct((B,), jnp.float32),
          jax.ShapeDtypeStruct((B * NEG,), jnp.float32),
      ),
      mesh=mesh,
      compiler_params=pltpu.CompilerParams(
          needs_layout_passes=False, use_tc_tiling_on_sc=True),
      scratch_types=[
          pltpu.VMEM((NCI,), jnp.int32),
          pltpu.VMEM((CHUNK,), jnp.int32),
          pltpu.VMEM((NNI,), jnp.int32),
          pltpu.VMEM((NCI,), jnp.int32),
          pltpu.VMEM((CHUNK,), jnp.int32),
          pltpu.VMEM((NNI,), jnp.int32),
          pltpu.VMEM((NCI, 128), jnp.float32),
          pltpu.VMEM((CHUNK, 128), jnp.float32),
          pltpu.VMEM((NNI, 128), jnp.float32),
          pltpu.VMEM((CHUNK,), jnp.float32),
          pltpu.VMEM((NNI,), jnp.float32),
          pltpu.SemaphoreType.DMA,
      ],
  )
  return kfn(ctx_idx, cen_idx, neg_idx, ctx_emb2, cen_emb2)


def kernel(context_words, center_words, negative_samples, context_emb,
           center_emb):
  ctx_idx = context_words.reshape(-1).astype(jnp.int32)
  cen_idx = center_words.astype(jnp.int32)
  neg_idx = negative_samples.reshape(-1).astype(jnp.int32)
  pos, neg = _cbow_sc(ctx_idx, cen_idx, neg_idx, context_emb.T, center_emb.T)
  return pos, neg.reshape(B, NEG)
